# baseline (speedup 1.0000x reference)
import functools

import jax
import jax.numpy as jnp
from jax import lax
from jax.experimental import pallas as pl
from jax.experimental.pallas import tpu as pltpu
from jax.experimental.pallas import tpu_sc as plsc

_B, _F, _V, _D = 4096, 26, 100000, 16
_H1, _H2, _H3 = 512, 256, 128


def _make_sc_gather_reduce():
    info = plsc.get_sparse_core_info()
    nc, ns = info.num_cores, info.num_subcores
    nw = nc * ns
    bpw = _B // nw

    mesh = plsc.VectorSubcoreMesh(core_axis_name="c", subcore_axis_name="s")

    @functools.partial(
        pl.kernel,
        mesh=mesh,
        compiler_params=pltpu.CompilerParams(use_tc_tiling_on_sc=False),
        out_type=(
            jax.ShapeDtypeStruct((_B, _D), jnp.float32),
            jax.ShapeDtypeStruct((_B, _D), jnp.float32),
        ),
        scratch_types=[
            pltpu.VMEM((_F, bpw), jnp.int32),
            pltpu.VMEM((_F, bpw, _D), jnp.float32),
            pltpu.VMEM((bpw, _D), jnp.float32),
            pltpu.VMEM((bpw, _D), jnp.float32),
            pltpu.SemaphoreType.DMA,
        ],
    )
    def sc_kernel(table_hbm, idx_hbm, sum_hbm, sumsq_hbm,
                  idx_v, rows_v, sum_v, sumsq_v, sem):
        wid = lax.axis_index("s") * nc + lax.axis_index("c")
        base = wid * bpw
        pltpu.sync_copy(idx_hbm.at[wid], idx_v)
        copies = [
            pltpu.async_copy(table_hbm.at[idx_v.at[f]], rows_v.at[f], sem)
            for f in range(_F)
        ]
        for c in copies:
            c.wait()

        def body(s, carry):
            r = rows_v[0, s, :]
            acc = r
            accsq = r * r
            for f in range(1, _F):
                r = rows_v[f, s, :]
                acc = acc + r
                accsq = accsq + r * r
            sum_v[s, :] = acc
            sumsq_v[s, :] = accsq
            return carry

        lax.fori_loop(0, bpw, body, 0)
        pltpu.sync_copy(sum_v, sum_hbm.at[pl.ds(base, bpw)])
        pltpu.sync_copy(sumsq_v, sumsq_hbm.at[pl.ds(base, bpw)])

    return sc_kernel, nw, bpw


def _mlp_body(sum_ref, sumsq_ref, pb_ref, w1, b1r, w2, b2r, w3, b3r, ws, bsr,
              out_ref):
    x = sum_ref[...]
    q = sumsq_ref[...]
    net = x + 0.5 * (x * x - q) + pb_ref[...]
    net = jnp.maximum(net, 0.0)
    h = jnp.dot(net, w1[...], preferred_element_type=jnp.float32) + b1r[...]
    h = jnp.maximum(h, 0.0)
    h = jnp.dot(h, w2[...], preferred_element_type=jnp.float32) + b2r[...]
    h = jnp.maximum(h, 0.0)
    h = jnp.dot(h, w3[...], preferred_element_type=jnp.float32) + b3r[...]
    h = jnp.maximum(h, 0.0)
    out_ref[...] = jnp.dot(h, ws[...], preferred_element_type=jnp.float32) + bsr[...]


def _mlp(sum_e, sumsq, pb, W1, b1, W2, b2, W3, b3, Ws, bs):
    blk = 1024
    grid = _B // blk
    full = lambda i: (0, 0)
    return pl.pallas_call(
        _mlp_body,
        grid=(grid,),
        in_specs=[
            pl.BlockSpec((blk, _D), lambda i: (i, 0)),
            pl.BlockSpec((blk, _D), lambda i: (i, 0)),
            pl.BlockSpec((1, _D), full),
            pl.BlockSpec((_D, _H1), full),
            pl.BlockSpec((1, _H1), full),
            pl.BlockSpec((_H1, _H2), full),
            pl.BlockSpec((1, _H2), full),
            pl.BlockSpec((_H2, _H3), full),
            pl.BlockSpec((1, _H3), full),
            pl.BlockSpec((_H3, 1), full),
            pl.BlockSpec((1, 1), full),
        ],
        out_specs=pl.BlockSpec((blk, 1), lambda i: (i, 0)),
        out_shape=jax.ShapeDtypeStruct((_B, 1), jnp.float32),
    )(sum_e, sumsq, pb.reshape(1, _D), W1, b1.reshape(1, _H1),
      W2, b2.reshape(1, _H2), W3, b3.reshape(1, _H3), Ws, bs.reshape(1, 1))


def kernel(indices, tables, product_bias, W1, b1, W2, b2, W3, b3, Ws, bs):
    sc_kernel, nw, bpw = _make_sc_gather_reduce()
    table_flat = tables.reshape(_F * _V, _D)
    off = (jnp.arange(_F, dtype=jnp.int32) * _V)[None, :]
    fi = (indices + off).T
    idx_blocks = fi.reshape(_F, nw, bpw).transpose(1, 0, 2)
    sum_e, sumsq = sc_kernel(table_flat, idx_blocks)
    return _mlp(sum_e, sumsq, product_bias, W1, b1, W2, b2, W3, b3, Ws, bs)


# speedup vs baseline: 2.9062x; 2.9062x over previous
import functools

import jax
import jax.numpy as jnp
from jax import lax
from jax.experimental import pallas as pl
from jax.experimental.pallas import tpu as pltpu
from jax.experimental.pallas import tpu_sc as plsc

_B, _F, _V, _D = 4096, 26, 100000, 16
_H1, _H2, _H3 = 512, 256, 128


def _make_sc_gather_reduce():
    info = plsc.get_sparse_core_info()
    nc, ns = info.num_cores, info.num_subcores
    nw = nc * ns
    bpw = _B // nw

    mesh = plsc.VectorSubcoreMesh(core_axis_name="c", subcore_axis_name="s")

    @functools.partial(
        pl.kernel,
        mesh=mesh,
        compiler_params=pltpu.CompilerParams(use_tc_tiling_on_sc=False),
        out_type=(
            jax.ShapeDtypeStruct((nw, _D, bpw), jnp.float32),
            jax.ShapeDtypeStruct((nw, _D, bpw), jnp.float32),
        ),
        scratch_types=[
            pltpu.VMEM((_F, bpw), jnp.int32),
            pltpu.VMEM((_D, bpw), jnp.int32),
            pltpu.VMEM((_D, bpw), jnp.float32),
            pltpu.VMEM((_D, bpw), jnp.float32),
            pltpu.VMEM((_D, bpw), jnp.float32),
            pltpu.SemaphoreType.DMA,
        ],
    )
    def sc_kernel(table_hbm, idx_hbm, sum_hbm, sumsq_hbm,
                  idx_v, idxf_v, rows_v, acc_v, accsq_v, sem):
        wid = lax.axis_index("s") * nc + lax.axis_index("c")
        pltpu.sync_copy(idx_hbm.at[wid], idx_v)

        zero = jnp.zeros((16,), jnp.float32)
        for d in range(_D):
            for c in range(0, bpw, 16):
                acc_v[d, c:c + 16] = zero
                accsq_v[d, c:c + 16] = zero

        def fbody(f, carry):
            for d in range(_D):
                o = (f * _D + d) * _V
                for c in range(0, bpw, 16):
                    idxf_v[d, c:c + 16] = idx_v[f, c:c + 16] + o
            copies = [
                pltpu.async_copy(table_hbm.at[idxf_v.at[d]], rows_v.at[d],
                                 sem)
                for d in range(_D)
            ]
            for cp in copies:
                cp.wait()
            for d in range(_D):
                for c in range(0, bpw, 16):
                    r = rows_v[d, c:c + 16]
                    acc_v[d, c:c + 16] = acc_v[d, c:c + 16] + r
                    accsq_v[d, c:c + 16] = accsq_v[d, c:c + 16] + r * r
            return carry

        lax.fori_loop(0, _F, fbody, 0)
        pltpu.sync_copy(acc_v, sum_hbm.at[wid])
        pltpu.sync_copy(accsq_v, sumsq_hbm.at[wid])

    return sc_kernel, nw, bpw


def _mlp_body(sum_ref, sumsq_ref, pb_ref, w1, b1r, w2, b2r, w3, b3r, ws, bsr,
              out_ref):
    x = sum_ref[...]
    q = sumsq_ref[...]
    net = x + 0.5 * (x * x - q) + pb_ref[...]
    net = jnp.maximum(net, 0.0)
    h = jnp.dot(net, w1[...], preferred_element_type=jnp.float32) + b1r[...]
    h = jnp.maximum(h, 0.0)
    h = jnp.dot(h, w2[...], preferred_element_type=jnp.float32) + b2r[...]
    h = jnp.maximum(h, 0.0)
    h = jnp.dot(h, w3[...], preferred_element_type=jnp.float32) + b3r[...]
    h = jnp.maximum(h, 0.0)
    out_ref[...] = jnp.dot(h, ws[...], preferred_element_type=jnp.float32) + bsr[...]


def _mlp(sum_e, sumsq, pb, W1, b1, W2, b2, W3, b3, Ws, bs):
    blk = 1024
    grid = _B // blk
    full = lambda i: (0, 0)
    return pl.pallas_call(
        _mlp_body,
        grid=(grid,),
        in_specs=[
            pl.BlockSpec((blk, _D), lambda i: (i, 0)),
            pl.BlockSpec((blk, _D), lambda i: (i, 0)),
            pl.BlockSpec((1, _D), full),
            pl.BlockSpec((_D, _H1), full),
            pl.BlockSpec((1, _H1), full),
            pl.BlockSpec((_H1, _H2), full),
            pl.BlockSpec((1, _H2), full),
            pl.BlockSpec((_H2, _H3), full),
            pl.BlockSpec((1, _H3), full),
            pl.BlockSpec((_H3, 1), full),
            pl.BlockSpec((1, 1), full),
        ],
        out_specs=pl.BlockSpec((blk, 1), lambda i: (i, 0)),
        out_shape=jax.ShapeDtypeStruct((_B, 1), jnp.float32),
    )(sum_e, sumsq, pb.reshape(1, _D), W1, b1.reshape(1, _H1),
      W2, b2.reshape(1, _H2), W3, b3.reshape(1, _H3), Ws, bs.reshape(1, 1))


def kernel(indices, tables, product_bias, W1, b1, W2, b2, W3, b3, Ws, bs):
    sc_kernel, nw, bpw = _make_sc_gather_reduce()
    table_flat = tables.transpose(0, 2, 1).reshape(_F * _D * _V)
    idx_blocks = indices.T.reshape(_F, nw, bpw).transpose(1, 0, 2)
    sum_w, sumsq_w = sc_kernel(table_flat, idx_blocks)
    sum_e = sum_w.transpose(0, 2, 1).reshape(_B, _D)
    sumsq = sumsq_w.transpose(0, 2, 1).reshape(_B, _D)
    return _mlp(sum_e, sumsq, product_bias, W1, b1, W2, b2, W3, b3, Ws, bs)
